# baseline (speedup 1.0000x reference)
import functools

import jax
import jax.numpy as jnp
import numpy as np
from jax.experimental import pallas as pl
from jax.experimental.pallas import tpu as pltpu

NUM_CB = 16
CB_DIM = 5
EFF = NUM_CB * CB_DIM
_HALF_L = (8 - 1.0) * (1.0 + 1e-3) / 2.0
_OFFSET = 0.5
_SHIFT = float(np.arctanh(_OFFSET / _HALF_L))
_HALF_W = 4.0

_TT = 1024


def _fsq_kernel(x_ref, win_ref, bin_ref, wout_ref, bout_ref,
                zq_ref, idx_ref, codes_scr):
    nt = pl.num_programs(1) - 1
    t = pl.program_id(1)

    @pl.when(t < nt)
    def _phase_in():
        z = jnp.dot(win_ref[...], x_ref[0],
                    preferred_element_type=jnp.float32)
        z = z + bin_ref[...]
        bounded = jnp.tanh(z + _SHIFT) * _HALF_L - _OFFSET
        rounded = jnp.round(bounded)
        codes_scr[:, pl.ds(t * _TT, _TT)] = rounded * (1.0 / _HALF_W)
        zhat = rounded + _HALF_W
        row = jax.lax.broadcasted_iota(jnp.int32, (NUM_CB, EFF), 0)
        col = jax.lax.broadcasted_iota(jnp.int32, (NUM_CB, EFF), 1)
        basis = jnp.exp2((3 * (col % CB_DIM)).astype(jnp.float32))
        sel = jnp.where(col // CB_DIM == row, basis, 0.0)
        idx = jnp.dot(sel, zhat, preferred_element_type=jnp.float32)
        idx_ref[0] = idx.astype(jnp.int32)

    @pl.when(t > 0)
    def _phase_out():
        tm1 = t - 1
        zq = jnp.dot(wout_ref[...], codes_scr[:, pl.ds(tm1 * _TT, _TT)],
                     preferred_element_type=jnp.float32)
        zq_ref[0] = zq + bout_ref[...]


@jax.jit
def _fsq_call(x, W_in, b_in, W_out, b_out):
    B, D, T = x.shape
    NT = T // _TT
    zq, idx = pl.pallas_call(
        _fsq_kernel,
        grid=(B, NT + 1),
        in_specs=[
            pl.BlockSpec((1, D, _TT),
                         lambda b, t: (b, 0, jnp.minimum(t, NT - 1))),
            pl.BlockSpec((EFF, D), lambda b, t: (0, 0)),
            pl.BlockSpec((EFF, 1), lambda b, t: (0, 0)),
            pl.BlockSpec((D, EFF), lambda b, t: (0, 0)),
            pl.BlockSpec((D, 1), lambda b, t: (0, 0)),
        ],
        out_specs=[
            pl.BlockSpec((1, D, _TT),
                         lambda b, t: (b, 0, jnp.maximum(t - 1, 0))),
            pl.BlockSpec((1, NUM_CB, _TT),
                         lambda b, t: (b, 0, jnp.minimum(t, NT - 1))),
        ],
        out_shape=[
            jax.ShapeDtypeStruct((B, D, T), jnp.float32),
            jax.ShapeDtypeStruct((B, NUM_CB, T), jnp.int32),
        ],
        scratch_shapes=[pltpu.VMEM((EFF, T), jnp.float32)],
        compiler_params=pltpu.CompilerParams(
            dimension_semantics=("arbitrary", "arbitrary"),
        ),
    )(x, W_in, b_in.reshape(EFF, 1), W_out, b_out.reshape(D, 1))
    return zq, idx


def kernel(x, W_in, b_in, W_out, b_out):
    zq, indices = _fsq_call(x, W_in, b_in, W_out, b_out)
    zero = jnp.zeros((), dtype=jnp.float32)
    return (zq, indices, None, zero, zero, zq)


# speedup vs baseline: 1.1496x; 1.1496x over previous
import functools

import jax
import jax.numpy as jnp
import numpy as np
from jax.experimental import pallas as pl
from jax.experimental.pallas import tpu as pltpu

NUM_CB = 16
CB_DIM = 5
EFF = NUM_CB * CB_DIM
_HALF_L = (8 - 1.0) * (1.0 + 1e-3) / 2.0
_OFFSET = 0.5
_SHIFT = float(np.arctanh(_OFFSET / _HALF_L))
_HALF_W = 4.0


def _fsq_kernel(x_ref, win_ref, bin_ref, wout_ref, bout_ref, zq_ref, idx_ref):
    z = jnp.dot(win_ref[...], x_ref[0], preferred_element_type=jnp.float32)
    z = z + bin_ref[...]
    bounded = jnp.tanh(z + _SHIFT) * _HALF_L - _OFFSET
    rounded = jnp.round(bounded)
    codes = rounded * (1.0 / _HALF_W)
    zq = jnp.dot(wout_ref[...], codes, preferred_element_type=jnp.float32)
    zq_ref[0] = zq + bout_ref[...]

    zhat = rounded + _HALF_W
    row = jax.lax.broadcasted_iota(jnp.int32, (NUM_CB, EFF), 0)
    col = jax.lax.broadcasted_iota(jnp.int32, (NUM_CB, EFF), 1)
    basis = jnp.exp2((3 * (col % CB_DIM)).astype(jnp.float32))
    sel = jnp.where(col // CB_DIM == row, basis, 0.0)
    idx = jnp.dot(sel, zhat, preferred_element_type=jnp.float32)
    idx_ref[0] = idx.astype(jnp.int32)


@jax.jit
def _fsq_call(x, W_in, b_in, W_out, b_out):
    B, D, T = x.shape
    TT = 1024
    grid = (B, T // TT)
    zq, idx = pl.pallas_call(
        _fsq_kernel,
        grid=grid,
        in_specs=[
            pl.BlockSpec((1, D, TT), lambda b, t: (b, 0, t)),
            pl.BlockSpec((EFF, D), lambda b, t: (0, 0)),
            pl.BlockSpec((EFF, 1), lambda b, t: (0, 0)),
            pl.BlockSpec((D, EFF), lambda b, t: (0, 0)),
            pl.BlockSpec((D, 1), lambda b, t: (0, 0)),
        ],
        out_specs=[
            pl.BlockSpec((1, D, TT), lambda b, t: (b, 0, t)),
            pl.BlockSpec((1, NUM_CB, TT), lambda b, t: (b, 0, t)),
        ],
        out_shape=[
            jax.ShapeDtypeStruct((B, D, T), jnp.float32),
            jax.ShapeDtypeStruct((B, NUM_CB, T), jnp.int32),
        ],
        compiler_params=pltpu.CompilerParams(
            dimension_semantics=("parallel", "parallel"),
        ),
    )(x, W_in, b_in.reshape(EFF, 1), W_out, b_out.reshape(D, 1))
    return zq, idx


def kernel(x, W_in, b_in, W_out, b_out):
    zq, indices = _fsq_call(x, W_in, b_in, W_out, b_out)
    zero = jnp.zeros((), dtype=jnp.float32)
    return (zq, indices, None, zero, zero, zq)
